# Initial kernel scaffold; baseline (speedup 1.0000x reference)
import functools

import jax
import jax.numpy as jnp
from jax import lax
from jax.experimental import pallas as pl
from jax.experimental.pallas import tpu as pltpu
from jax.experimental.pallas import tpu_sc as plsc

N = 10000
E = 320000
D = 128
H = 64

NC = 2
NS = 16
EPC = E // NC
EPW = E // (NC * NS)
CHUNK = 80
NCHUNK = EPW // CHUNK
ROWS_PW = N // NS
CW = 16

_f32 = jnp.float32
_sc_mesh = plsc.VectorSubcoreMesh(core_axis_name="c", subcore_axis_name="s")


def _sc_body(with_cnt, *refs):
    if with_cnt:
        (src_hbm, dst_hbm, p_hbm, z64_hbm, z16_hbm, ones_hbm,
         out_s, out_c,
         acc, cntacc, src_v, dst_v, rows_v, ones_v, sem) = refs
    else:
        (src_hbm, dst_hbm, p_hbm, z64_hbm,
         out_s,
         acc, src_v, dst_v, rows_v, sem) = refs
    c = lax.axis_index("c")
    s = lax.axis_index("s")
    rbase = s * ROWS_PW
    pltpu.sync_copy(z64_hbm.at[pl.ds(rbase, ROWS_PW)],
                    acc.at[pl.ds(rbase, ROWS_PW)])
    if with_cnt:
        pltpu.sync_copy(z16_hbm.at[pl.ds(rbase, ROWS_PW)],
                        cntacc.at[pl.ds(rbase, ROWS_PW)])
        pltpu.sync_copy(ones_hbm, ones_v)
    plsc.subcore_barrier()

    ebase = c * EPC + s * EPW

    @pl.loop(0, NCHUNK)
    def _(j):
        base = ebase + j * CHUNK
        pltpu.sync_copy(src_hbm.at[pl.ds(base, CHUNK)], src_v)
        pltpu.sync_copy(dst_hbm.at[pl.ds(base, CHUNK)], dst_v)
        pltpu.async_copy(p_hbm.at[src_v], rows_v, sem).wait()
        pltpu.sync_copy(rows_v, acc.at[dst_v], add=True)
        if with_cnt:
            pltpu.sync_copy(ones_v, cntacc.at[dst_v], add=True)

    plsc.subcore_barrier()
    pltpu.sync_copy(acc.at[pl.ds(rbase, ROWS_PW)],
                    out_s.at[c, pl.ds(rbase, ROWS_PW)])
    if with_cnt:
        pltpu.sync_copy(cntacc.at[pl.ds(rbase, ROWS_PW)],
                        out_c.at[c, pl.ds(rbase, ROWS_PW)])


_sc_seg_sum_cnt = pl.kernel(
    functools.partial(_sc_body, True),
    out_type=[jax.ShapeDtypeStruct((NC, N, H), _f32),
              jax.ShapeDtypeStruct((NC, N, CW), _f32)],
    mesh=_sc_mesh,
    scratch_types=[
        pltpu.VMEM_SHARED((N, H), _f32),
        pltpu.VMEM_SHARED((N, CW), _f32),
        pltpu.VMEM((CHUNK,), jnp.int32),
        pltpu.VMEM((CHUNK,), jnp.int32),
        pltpu.VMEM((CHUNK, H), _f32),
        pltpu.VMEM((CHUNK, CW), _f32),
        pltpu.SemaphoreType.DMA,
    ],
)

_sc_seg_sum = pl.kernel(
    functools.partial(_sc_body, False),
    out_type=jax.ShapeDtypeStruct((NC, N, H), _f32),
    mesh=_sc_mesh,
    scratch_types=[
        pltpu.VMEM_SHARED((N, H), _f32),
        pltpu.VMEM((CHUNK,), jnp.int32),
        pltpu.VMEM((CHUNK,), jnp.int32),
        pltpu.VMEM((CHUNK, H), _f32),
        pltpu.SemaphoreType.DMA,
    ],
)


_BLK = 2000
_GRID = N // _BLK


def _pre0_body(x_ref, wl_ref, wr_ref, wp_ref, p_ref, r_ref, res_ref):
    xb = x_ref[...]
    p_ref[...] = jnp.dot(xb, wl_ref[...], preferred_element_type=_f32)
    r_ref[...] = jnp.dot(xb, wr_ref[...], preferred_element_type=_f32)
    res_ref[...] = jnp.dot(xb, wp_ref[...], preferred_element_type=_f32)


_pre0 = pl.pallas_call(
    _pre0_body,
    grid=(_GRID,),
    in_specs=[
        pl.BlockSpec((_BLK, D), lambda i: (i, 0)),
        pl.BlockSpec((D, H), lambda i: (0, 0)),
        pl.BlockSpec((D, H), lambda i: (0, 0)),
        pl.BlockSpec((D, H), lambda i: (0, 0)),
    ],
    out_specs=[
        pl.BlockSpec((_BLK, H), lambda i: (i, 0)),
        pl.BlockSpec((_BLK, H), lambda i: (i, 0)),
        pl.BlockSpec((_BLK, H), lambda i: (i, 0)),
    ],
    out_shape=[jax.ShapeDtypeStruct((N, H), _f32)] * 3,
)


def _combine(s_ref, c_ref, r_ref, bl_ref, g_ref, b_ref, res_ref):
    ssum = s_ref[0] + s_ref[1]
    cnt = c_ref[0, :, 0:1] + c_ref[1, :, 0:1]
    agg = ssum / jnp.maximum(cnt, 1.0)
    z = agg + bl_ref[...] + r_ref[...]
    mu = jnp.mean(z, axis=-1, keepdims=True)
    d = z - mu
    var = jnp.mean(d * d, axis=-1, keepdims=True)
    zn = d * lax.rsqrt(var + 1e-5) * g_ref[...] + b_ref[...]
    return jnp.maximum(zn, 0.0) + res_ref[...]


def _post_mid_body(s_ref, c_ref, r_ref, bl_ref, g_ref, b_ref, res_ref,
                   wln_ref, wrn_ref, h_ref, pn_ref, rn_ref):
    h = _combine(s_ref, c_ref, r_ref, bl_ref, g_ref, b_ref, res_ref)
    h_ref[...] = h
    pn_ref[...] = jnp.dot(h, wln_ref[...], preferred_element_type=_f32)
    rn_ref[...] = jnp.dot(h, wrn_ref[...], preferred_element_type=_f32)


_post_mid = pl.pallas_call(
    _post_mid_body,
    grid=(_GRID,),
    in_specs=[
        pl.BlockSpec((NC, _BLK, H), lambda i: (0, i, 0)),
        pl.BlockSpec((NC, _BLK, CW), lambda i: (0, i, 0)),
        pl.BlockSpec((_BLK, H), lambda i: (i, 0)),
        pl.BlockSpec((1, H), lambda i: (0, 0)),
        pl.BlockSpec((1, H), lambda i: (0, 0)),
        pl.BlockSpec((1, H), lambda i: (0, 0)),
        pl.BlockSpec((_BLK, H), lambda i: (i, 0)),
        pl.BlockSpec((H, H), lambda i: (0, 0)),
        pl.BlockSpec((H, H), lambda i: (0, 0)),
    ],
    out_specs=[
        pl.BlockSpec((_BLK, H), lambda i: (i, 0)),
        pl.BlockSpec((_BLK, H), lambda i: (i, 0)),
        pl.BlockSpec((_BLK, H), lambda i: (i, 0)),
    ],
    out_shape=[jax.ShapeDtypeStruct((N, H), _f32)] * 3,
)


def _post_last_body(s_ref, c_ref, r_ref, bl_ref, g_ref, b_ref, res_ref,
                    wh_ref, bh_ref, h_ref, y_ref):
    h = _combine(s_ref, c_ref, r_ref, bl_ref, g_ref, b_ref, res_ref)
    h_ref[...] = h
    y_ref[...] = jnp.dot(h, wh_ref[...], preferred_element_type=_f32) + bh_ref[...]


_post_last = pl.pallas_call(
    _post_last_body,
    grid=(_GRID,),
    in_specs=[
        pl.BlockSpec((NC, _BLK, H), lambda i: (0, i, 0)),
        pl.BlockSpec((NC, _BLK, CW), lambda i: (0, i, 0)),
        pl.BlockSpec((_BLK, H), lambda i: (i, 0)),
        pl.BlockSpec((1, H), lambda i: (0, 0)),
        pl.BlockSpec((1, H), lambda i: (0, 0)),
        pl.BlockSpec((1, H), lambda i: (0, 0)),
        pl.BlockSpec((_BLK, H), lambda i: (i, 0)),
        pl.BlockSpec((H, 1), lambda i: (0, 0)),
        pl.BlockSpec((1, 1), lambda i: (0, 0)),
    ],
    out_specs=[
        pl.BlockSpec((_BLK, H), lambda i: (i, 0)),
        pl.BlockSpec((_BLK, 1), lambda i: (i, 0)),
    ],
    out_shape=[jax.ShapeDtypeStruct((N, H), _f32),
               jax.ShapeDtypeStruct((N, 1), _f32)],
)


def kernel(x, edge_index, Wl0, bl0, Wr0, ln_g0, ln_b0, Wl1, bl1, Wr1,
           ln_g1, ln_b1, Wl2, bl2, Wr2, ln_g2, ln_b2, Wproj, Whead, bhead):
    src = edge_index[0]
    dst = edge_index[1]
    z64 = jnp.zeros((N, H), _f32)
    z16 = jnp.zeros((N, CW), _f32)
    ones = jnp.ones((CHUNK, CW), _f32)

    bl0r, g0r, b0r = bl0.reshape(1, H), ln_g0.reshape(1, H), ln_b0.reshape(1, H)
    bl1r, g1r, b1r = bl1.reshape(1, H), ln_g1.reshape(1, H), ln_b1.reshape(1, H)
    bl2r, g2r, b2r = bl2.reshape(1, H), ln_g2.reshape(1, H), ln_b2.reshape(1, H)

    p0, r0, res0 = _pre0(x, Wl0.T, Wr0.T, Wproj.T)
    s0, cpart = _sc_seg_sum_cnt(src, dst, p0, z64, z16, ones)
    h1, p1, r1 = _post_mid(s0, cpart, r0, bl0r, g0r, b0r, res0, Wl1.T, Wr1.T)
    s1 = _sc_seg_sum(src, dst, p1, z64)
    h2, p2, r2 = _post_mid(s1, cpart, r1, bl1r, g1r, b1r, h1, Wl2.T, Wr2.T)
    s2 = _sc_seg_sum(src, dst, p2, z64)
    h3, y = _post_last(s2, cpart, r2, bl2r, g2r, b2r, h2,
                       Whead.T, bhead.reshape(1, 1))
    return (y[:, 0], h3)


# speedup vs baseline: 5.1221x; 5.1221x over previous
import functools

import jax
import jax.numpy as jnp
from jax import lax
from jax.experimental import pallas as pl
from jax.experimental.pallas import tpu as pltpu
from jax.experimental.pallas import tpu_sc as plsc

N = 10000
E = 320000
D = 128
H = 64

NC = 2
NS = 16
EPC = E // NC
EPW = E // (NC * NS)
CHUNK = 80
NCHUNK = EPW // CHUNK
N_PAD = 10240
ROWS_PW = N_PAD // NS
CW = 16

_f32 = jnp.float32
_sc_mesh = plsc.VectorSubcoreMesh(core_axis_name="c", subcore_axis_name="s")


def _sc_body(with_cnt, *refs):
    if with_cnt:
        (src_hbm, dst_hbm, p_hbm, z64_hbm, z16_hbm, ones_hbm,
         out_s, out_c,
         acc, cntacc, src_v, dst_v, rows_v, ones_v, sem) = refs
    else:
        (src_hbm, dst_hbm, p_hbm, z64_hbm,
         out_s,
         acc, src_v, dst_v, rows_v, sem) = refs
    c = lax.axis_index("c")
    s = lax.axis_index("s")
    rbase = s * ROWS_PW
    pltpu.sync_copy(z64_hbm.at[pl.ds(rbase, ROWS_PW)],
                    acc.at[pl.ds(rbase, ROWS_PW)])
    if with_cnt:
        pltpu.sync_copy(z16_hbm.at[pl.ds(rbase, ROWS_PW)],
                        cntacc.at[pl.ds(rbase, ROWS_PW)])
        pltpu.sync_copy(ones_hbm, ones_v)
    plsc.subcore_barrier()

    ebase = c * EPC + s * EPW

    @pl.loop(0, NCHUNK)
    def _(j):
        base = ebase + j * CHUNK
        pltpu.sync_copy(src_hbm.at[pl.ds(base, CHUNK)], src_v)
        pltpu.sync_copy(dst_hbm.at[pl.ds(base, CHUNK)], dst_v)
        pltpu.async_copy(p_hbm.at[src_v], rows_v, sem).wait()
        pltpu.sync_copy(rows_v, acc.at[dst_v], add=True)
        if with_cnt:
            pltpu.sync_copy(ones_v, cntacc.at[dst_v], add=True)

    plsc.subcore_barrier()
    pltpu.sync_copy(acc.at[pl.ds(rbase, ROWS_PW)],
                    out_s.at[c, pl.ds(rbase, ROWS_PW)])
    if with_cnt:
        pltpu.sync_copy(cntacc.at[pl.ds(rbase, ROWS_PW)],
                        out_c.at[c, pl.ds(rbase, ROWS_PW)])


_sc_seg_sum_cnt = pl.kernel(
    functools.partial(_sc_body, True),
    out_type=[jax.ShapeDtypeStruct((NC, N_PAD, H), _f32),
              jax.ShapeDtypeStruct((NC, N_PAD, CW), _f32)],
    mesh=_sc_mesh,
    compiler_params=pltpu.CompilerParams(use_tc_tiling_on_sc=False),
    scratch_types=[
        pltpu.VMEM_SHARED((N_PAD, H), _f32),
        pltpu.VMEM_SHARED((N_PAD, CW), _f32),
        pltpu.VMEM((CHUNK,), jnp.int32),
        pltpu.VMEM((CHUNK,), jnp.int32),
        pltpu.VMEM((CHUNK, H), _f32),
        pltpu.VMEM((CHUNK, CW), _f32),
        pltpu.SemaphoreType.DMA,
    ],
)

_sc_seg_sum = pl.kernel(
    functools.partial(_sc_body, False),
    out_type=jax.ShapeDtypeStruct((NC, N_PAD, H), _f32),
    mesh=_sc_mesh,
    compiler_params=pltpu.CompilerParams(use_tc_tiling_on_sc=False),
    scratch_types=[
        pltpu.VMEM_SHARED((N_PAD, H), _f32),
        pltpu.VMEM((CHUNK,), jnp.int32),
        pltpu.VMEM((CHUNK,), jnp.int32),
        pltpu.VMEM((CHUNK, H), _f32),
        pltpu.SemaphoreType.DMA,
    ],
)


_BLK = 2000
_GRID = N // _BLK


def _pre0_body(x_ref, wl_ref, wr_ref, wp_ref, p_ref, r_ref, res_ref):
    xb = x_ref[...]
    p_ref[...] = jnp.dot(xb, wl_ref[...], preferred_element_type=_f32)
    r_ref[...] = jnp.dot(xb, wr_ref[...], preferred_element_type=_f32)
    res_ref[...] = jnp.dot(xb, wp_ref[...], preferred_element_type=_f32)


_pre0 = pl.pallas_call(
    _pre0_body,
    grid=(_GRID,),
    in_specs=[
        pl.BlockSpec((_BLK, D), lambda i: (i, 0)),
        pl.BlockSpec((D, H), lambda i: (0, 0)),
        pl.BlockSpec((D, H), lambda i: (0, 0)),
        pl.BlockSpec((D, H), lambda i: (0, 0)),
    ],
    out_specs=[
        pl.BlockSpec((_BLK, H), lambda i: (i, 0)),
        pl.BlockSpec((_BLK, H), lambda i: (i, 0)),
        pl.BlockSpec((_BLK, H), lambda i: (i, 0)),
    ],
    out_shape=[jax.ShapeDtypeStruct((N, H), _f32)] * 3,
)


def _combine(s_ref, c_ref, r_ref, bl_ref, g_ref, b_ref, res_ref):
    ssum = s_ref[0] + s_ref[1]
    cnt = c_ref[0, :, 0:1] + c_ref[1, :, 0:1]
    agg = ssum / jnp.maximum(cnt, 1.0)
    z = agg + bl_ref[...] + r_ref[...]
    mu = jnp.mean(z, axis=-1, keepdims=True)
    d = z - mu
    var = jnp.mean(d * d, axis=-1, keepdims=True)
    zn = d * lax.rsqrt(var + 1e-5) * g_ref[...] + b_ref[...]
    return jnp.maximum(zn, 0.0) + res_ref[...]


def _post_mid_body(s_ref, c_ref, r_ref, bl_ref, g_ref, b_ref, res_ref,
                   wln_ref, wrn_ref, h_ref, pn_ref, rn_ref):
    h = _combine(s_ref, c_ref, r_ref, bl_ref, g_ref, b_ref, res_ref)
    h_ref[...] = h
    pn_ref[...] = jnp.dot(h, wln_ref[...], preferred_element_type=_f32)
    rn_ref[...] = jnp.dot(h, wrn_ref[...], preferred_element_type=_f32)


_post_mid = pl.pallas_call(
    _post_mid_body,
    grid=(_GRID,),
    in_specs=[
        pl.BlockSpec((NC, _BLK, H), lambda i: (0, i, 0)),
        pl.BlockSpec((NC, _BLK, CW), lambda i: (0, i, 0)),
        pl.BlockSpec((_BLK, H), lambda i: (i, 0)),
        pl.BlockSpec((1, H), lambda i: (0, 0)),
        pl.BlockSpec((1, H), lambda i: (0, 0)),
        pl.BlockSpec((1, H), lambda i: (0, 0)),
        pl.BlockSpec((_BLK, H), lambda i: (i, 0)),
        pl.BlockSpec((H, H), lambda i: (0, 0)),
        pl.BlockSpec((H, H), lambda i: (0, 0)),
    ],
    out_specs=[
        pl.BlockSpec((_BLK, H), lambda i: (i, 0)),
        pl.BlockSpec((_BLK, H), lambda i: (i, 0)),
        pl.BlockSpec((_BLK, H), lambda i: (i, 0)),
    ],
    out_shape=[jax.ShapeDtypeStruct((N, H), _f32)] * 3,
)


def _post_last_body(s_ref, c_ref, r_ref, bl_ref, g_ref, b_ref, res_ref,
                    wh_ref, bh_ref, h_ref, y_ref):
    h = _combine(s_ref, c_ref, r_ref, bl_ref, g_ref, b_ref, res_ref)
    h_ref[...] = h
    y_ref[...] = jnp.dot(h, wh_ref[...], preferred_element_type=_f32) + bh_ref[...]


_post_last = pl.pallas_call(
    _post_last_body,
    grid=(_GRID,),
    in_specs=[
        pl.BlockSpec((NC, _BLK, H), lambda i: (0, i, 0)),
        pl.BlockSpec((NC, _BLK, CW), lambda i: (0, i, 0)),
        pl.BlockSpec((_BLK, H), lambda i: (i, 0)),
        pl.BlockSpec((1, H), lambda i: (0, 0)),
        pl.BlockSpec((1, H), lambda i: (0, 0)),
        pl.BlockSpec((1, H), lambda i: (0, 0)),
        pl.BlockSpec((_BLK, H), lambda i: (i, 0)),
        pl.BlockSpec((H, 1), lambda i: (0, 0)),
        pl.BlockSpec((1, 1), lambda i: (0, 0)),
    ],
    out_specs=[
        pl.BlockSpec((_BLK, H), lambda i: (i, 0)),
        pl.BlockSpec((_BLK, 1), lambda i: (i, 0)),
    ],
    out_shape=[jax.ShapeDtypeStruct((N, H), _f32),
               jax.ShapeDtypeStruct((N, 1), _f32)],
)


def kernel(x, edge_index, Wl0, bl0, Wr0, ln_g0, ln_b0, Wl1, bl1, Wr1,
           ln_g1, ln_b1, Wl2, bl2, Wr2, ln_g2, ln_b2, Wproj, Whead, bhead):
    src = edge_index[0]
    dst = edge_index[1]
    z64 = jnp.zeros((N_PAD, H), _f32)
    z16 = jnp.zeros((N_PAD, CW), _f32)
    ones = jnp.ones((CHUNK, CW), _f32)

    bl0r, g0r, b0r = bl0.reshape(1, H), ln_g0.reshape(1, H), ln_b0.reshape(1, H)
    bl1r, g1r, b1r = bl1.reshape(1, H), ln_g1.reshape(1, H), ln_b1.reshape(1, H)
    bl2r, g2r, b2r = bl2.reshape(1, H), ln_g2.reshape(1, H), ln_b2.reshape(1, H)

    p0, r0, res0 = _pre0(x, Wl0.T, Wr0.T, Wproj.T)
    s0, cpart = _sc_seg_sum_cnt(src, dst, p0, z64, z16, ones)
    h1, p1, r1 = _post_mid(s0, cpart, r0, bl0r, g0r, b0r, res0, Wl1.T, Wr1.T)
    s1 = _sc_seg_sum(src, dst, p1, z64)
    h2, p2, r2 = _post_mid(s1, cpart, r1, bl1r, g1r, b1r, h1, Wl2.T, Wr2.T)
    s2 = _sc_seg_sum(src, dst, p2, z64)
    h3, y = _post_last(s2, cpart, r2, bl2r, g2r, b2r, h2,
                       Whead.T, bhead.reshape(1, 1))
    return (y[:, 0], h3)


# speedup vs baseline: 14.8011x; 2.8897x over previous
import functools

import jax
import jax.numpy as jnp
from jax import lax
from jax.experimental import pallas as pl
from jax.experimental.pallas import tpu as pltpu
from jax.experimental.pallas import tpu_sc as plsc

N = 10000
E = 320000
D = 128
H = 64

NC = 2
NS = 16
EPC = E // NC
EPW = E // (NC * NS)
CHUNK = 80
NCHUNK = EPW // CHUNK
NBUF = 5
N_PAD = 10240
ROWS_PW = N_PAD // NS
CW = 16

_f32 = jnp.float32
_sc_mesh = plsc.VectorSubcoreMesh(core_axis_name="c", subcore_axis_name="s")


def _sc_body(with_cnt, *refs):
    if with_cnt:
        (src_hbm, dst_hbm, p_hbm, z64_hbm, z16_hbm, ones_hbm,
         out_s, out_c,
         acc, cntacc, src_v, dst_v, ones_v) = refs[:13]
        rows = refs[13:13 + NBUF]
        sems = refs[13 + NBUF:13 + 2 * NBUF]
    else:
        (src_hbm, dst_hbm, p_hbm, z64_hbm,
         out_s,
         acc, src_v, dst_v) = refs[:8]
        rows = refs[8:8 + NBUF]
        sems = refs[8 + NBUF:8 + 2 * NBUF]
    c = lax.axis_index("c")
    s = lax.axis_index("s")
    rbase = s * ROWS_PW
    pltpu.sync_copy(z64_hbm.at[pl.ds(rbase, ROWS_PW)],
                    acc.at[pl.ds(rbase, ROWS_PW)])
    pltpu.sync_copy(src_hbm.at[c, s], src_v)
    pltpu.sync_copy(dst_hbm.at[c, s], dst_v)
    if with_cnt:
        pltpu.sync_copy(z16_hbm.at[pl.ds(rbase, ROWS_PW)],
                        cntacc.at[pl.ds(rbase, ROWS_PW)])
        pltpu.sync_copy(ones_hbm, ones_v)
    plsc.subcore_barrier()

    for b in range(NBUF):
        pltpu.async_copy(p_hbm.at[src_v.at[b]], rows[b], sems[b])

    def _process(j, b, issue_next):
        pltpu.make_async_copy(p_hbm.at[src_v.at[j]], rows[b], sems[b]).wait()
        pltpu.sync_copy(rows[b], acc.at[dst_v.at[j]], add=True)
        if with_cnt:
            pltpu.sync_copy(ones_v, cntacc.at[dst_v.at[j]], add=True)
        if issue_next:
            pltpu.async_copy(p_hbm.at[src_v.at[j + NBUF]], rows[b], sems[b])

    @pl.loop(0, NCHUNK - NBUF, step=NBUF)
    def _(g):
        for b in range(NBUF):
            _process(g + b, b, True)

    for b in range(NBUF):
        _process(NCHUNK - NBUF + b, b, False)

    plsc.subcore_barrier()
    pltpu.sync_copy(acc.at[pl.ds(rbase, ROWS_PW)],
                    out_s.at[c, pl.ds(rbase, ROWS_PW)])
    if with_cnt:
        pltpu.sync_copy(cntacc.at[pl.ds(rbase, ROWS_PW)],
                        out_c.at[c, pl.ds(rbase, ROWS_PW)])


_sc_seg_sum_cnt = pl.kernel(
    functools.partial(_sc_body, True),
    out_type=[jax.ShapeDtypeStruct((NC, N_PAD, H), _f32),
              jax.ShapeDtypeStruct((NC, N_PAD, CW), _f32)],
    mesh=_sc_mesh,
    compiler_params=pltpu.CompilerParams(use_tc_tiling_on_sc=False),
    scratch_types=(
        [pltpu.VMEM_SHARED((N_PAD, H), _f32),
         pltpu.VMEM_SHARED((N_PAD, CW), _f32),
         pltpu.VMEM((NCHUNK, CHUNK), jnp.int32),
         pltpu.VMEM((NCHUNK, CHUNK), jnp.int32),
         pltpu.VMEM((CHUNK, CW), _f32)]
        + [pltpu.VMEM((CHUNK, H), _f32)] * NBUF
        + [pltpu.SemaphoreType.DMA] * NBUF
    ),
)

_sc_seg_sum = pl.kernel(
    functools.partial(_sc_body, False),
    out_type=jax.ShapeDtypeStruct((NC, N_PAD, H), _f32),
    mesh=_sc_mesh,
    compiler_params=pltpu.CompilerParams(use_tc_tiling_on_sc=False),
    scratch_types=(
        [pltpu.VMEM_SHARED((N_PAD, H), _f32),
         pltpu.VMEM((NCHUNK, CHUNK), jnp.int32),
         pltpu.VMEM((NCHUNK, CHUNK), jnp.int32)]
        + [pltpu.VMEM((CHUNK, H), _f32)] * NBUF
        + [pltpu.SemaphoreType.DMA] * NBUF
    ),
)


_BLK = 2000
_GRID = N // _BLK


def _pre0_body(x_ref, wl_ref, wr_ref, wp_ref, p_ref, r_ref, res_ref):
    xb = x_ref[...]
    p_ref[...] = jnp.dot(xb, wl_ref[...], preferred_element_type=_f32)
    r_ref[...] = jnp.dot(xb, wr_ref[...], preferred_element_type=_f32)
    res_ref[...] = jnp.dot(xb, wp_ref[...], preferred_element_type=_f32)


_pre0 = pl.pallas_call(
    _pre0_body,
    grid=(_GRID,),
    in_specs=[
        pl.BlockSpec((_BLK, D), lambda i: (i, 0)),
        pl.BlockSpec((D, H), lambda i: (0, 0)),
        pl.BlockSpec((D, H), lambda i: (0, 0)),
        pl.BlockSpec((D, H), lambda i: (0, 0)),
    ],
    out_specs=[
        pl.BlockSpec((_BLK, H), lambda i: (i, 0)),
        pl.BlockSpec((_BLK, H), lambda i: (i, 0)),
        pl.BlockSpec((_BLK, H), lambda i: (i, 0)),
    ],
    out_shape=[jax.ShapeDtypeStruct((N, H), _f32)] * 3,
)


def _combine(s_ref, c_ref, r_ref, bl_ref, g_ref, b_ref, res_ref):
    ssum = s_ref[0] + s_ref[1]
    cnt = c_ref[0, :, 0:1] + c_ref[1, :, 0:1]
    agg = ssum / jnp.maximum(cnt, 1.0)
    z = agg + bl_ref[...] + r_ref[...]
    mu = jnp.mean(z, axis=-1, keepdims=True)
    d = z - mu
    var = jnp.mean(d * d, axis=-1, keepdims=True)
    zn = d * lax.rsqrt(var + 1e-5) * g_ref[...] + b_ref[...]
    return jnp.maximum(zn, 0.0) + res_ref[...]


def _post_mid_body(s_ref, c_ref, r_ref, bl_ref, g_ref, b_ref, res_ref,
                   wln_ref, wrn_ref, h_ref, pn_ref, rn_ref):
    h = _combine(s_ref, c_ref, r_ref, bl_ref, g_ref, b_ref, res_ref)
    h_ref[...] = h
    pn_ref[...] = jnp.dot(h, wln_ref[...], preferred_element_type=_f32)
    rn_ref[...] = jnp.dot(h, wrn_ref[...], preferred_element_type=_f32)


_post_mid = pl.pallas_call(
    _post_mid_body,
    grid=(_GRID,),
    in_specs=[
        pl.BlockSpec((NC, _BLK, H), lambda i: (0, i, 0)),
        pl.BlockSpec((NC, _BLK, CW), lambda i: (0, i, 0)),
        pl.BlockSpec((_BLK, H), lambda i: (i, 0)),
        pl.BlockSpec((1, H), lambda i: (0, 0)),
        pl.BlockSpec((1, H), lambda i: (0, 0)),
        pl.BlockSpec((1, H), lambda i: (0, 0)),
        pl.BlockSpec((_BLK, H), lambda i: (i, 0)),
        pl.BlockSpec((H, H), lambda i: (0, 0)),
        pl.BlockSpec((H, H), lambda i: (0, 0)),
    ],
    out_specs=[
        pl.BlockSpec((_BLK, H), lambda i: (i, 0)),
        pl.BlockSpec((_BLK, H), lambda i: (i, 0)),
        pl.BlockSpec((_BLK, H), lambda i: (i, 0)),
    ],
    out_shape=[jax.ShapeDtypeStruct((N, H), _f32)] * 3,
)


def _post_last_body(s_ref, c_ref, r_ref, bl_ref, g_ref, b_ref, res_ref,
                    wh_ref, bh_ref, h_ref, y_ref):
    h = _combine(s_ref, c_ref, r_ref, bl_ref, g_ref, b_ref, res_ref)
    h_ref[...] = h
    y_ref[...] = jnp.dot(h, wh_ref[...], preferred_element_type=_f32) + bh_ref[...]


_post_last = pl.pallas_call(
    _post_last_body,
    grid=(_GRID,),
    in_specs=[
        pl.BlockSpec((NC, _BLK, H), lambda i: (0, i, 0)),
        pl.BlockSpec((NC, _BLK, CW), lambda i: (0, i, 0)),
        pl.BlockSpec((_BLK, H), lambda i: (i, 0)),
        pl.BlockSpec((1, H), lambda i: (0, 0)),
        pl.BlockSpec((1, H), lambda i: (0, 0)),
        pl.BlockSpec((1, H), lambda i: (0, 0)),
        pl.BlockSpec((_BLK, H), lambda i: (i, 0)),
        pl.BlockSpec((H, 1), lambda i: (0, 0)),
        pl.BlockSpec((1, 1), lambda i: (0, 0)),
    ],
    out_specs=[
        pl.BlockSpec((_BLK, H), lambda i: (i, 0)),
        pl.BlockSpec((_BLK, 1), lambda i: (i, 0)),
    ],
    out_shape=[jax.ShapeDtypeStruct((N, H), _f32),
               jax.ShapeDtypeStruct((N, 1), _f32)],
)


def kernel(x, edge_index, Wl0, bl0, Wr0, ln_g0, ln_b0, Wl1, bl1, Wr1,
           ln_g1, ln_b1, Wl2, bl2, Wr2, ln_g2, ln_b2, Wproj, Whead, bhead):
    src = edge_index[0].reshape(NC, NS, NCHUNK, CHUNK)
    dst = edge_index[1].reshape(NC, NS, NCHUNK, CHUNK)
    z64 = jnp.zeros((N_PAD, H), _f32)
    z16 = jnp.zeros((N_PAD, CW), _f32)
    ones = jnp.ones((CHUNK, CW), _f32)

    bl0r, g0r, b0r = bl0.reshape(1, H), ln_g0.reshape(1, H), ln_b0.reshape(1, H)
    bl1r, g1r, b1r = bl1.reshape(1, H), ln_g1.reshape(1, H), ln_b1.reshape(1, H)
    bl2r, g2r, b2r = bl2.reshape(1, H), ln_g2.reshape(1, H), ln_b2.reshape(1, H)

    p0, r0, res0 = _pre0(x, Wl0.T, Wr0.T, Wproj.T)
    s0, cpart = _sc_seg_sum_cnt(src, dst, p0, z64, z16, ones)
    h1, p1, r1 = _post_mid(s0, cpart, r0, bl0r, g0r, b0r, res0, Wl1.T, Wr1.T)
    s1 = _sc_seg_sum(src, dst, p1, z64)
    h2, p2, r2 = _post_mid(s1, cpart, r1, bl1r, g1r, b1r, h1, Wl2.T, Wr2.T)
    s2 = _sc_seg_sum(src, dst, p2, z64)
    h3, y = _post_last(s2, cpart, r2, bl2r, g2r, b2r, h2,
                       Whead.T, bhead.reshape(1, 1))
    return (y[:, 0], h3)


# speedup vs baseline: 14.8744x; 1.0050x over previous
import functools

import jax
import jax.numpy as jnp
from jax import lax
from jax.experimental import pallas as pl
from jax.experimental.pallas import tpu as pltpu
from jax.experimental.pallas import tpu_sc as plsc

N = 10000
E = 320000
D = 128
H = 64

NC = 2
NS = 16
EPC = E // NC
EPW = E // (NC * NS)
CHUNK = 80
NCHUNK = EPW // CHUNK
NBUF = 5
N_PAD = 10240
ROWS_PW = N_PAD // NS
CW = 16

_f32 = jnp.float32
_sc_mesh = plsc.VectorSubcoreMesh(core_axis_name="c", subcore_axis_name="s")


def _sc_body(with_cnt, *refs):
    if with_cnt:
        (src_hbm, dst_hbm, p_hbm, z64_hbm, z16_hbm, ones_hbm,
         out_s, out_c,
         acc, cntacc, src_v, dst_v, ones_v) = refs[:13]
        rows = refs[13:13 + NBUF]
        sems = refs[13 + NBUF:13 + 2 * NBUF]
    else:
        (src_hbm, dst_hbm, p_hbm, z64_hbm,
         out_s,
         acc, src_v, dst_v) = refs[:8]
        rows = refs[8:8 + NBUF]
        sems = refs[8 + NBUF:8 + 2 * NBUF]
    c = lax.axis_index("c")
    s = lax.axis_index("s")
    rbase = s * ROWS_PW
    pltpu.sync_copy(z64_hbm.at[pl.ds(rbase, ROWS_PW)],
                    acc.at[pl.ds(rbase, ROWS_PW)])
    pltpu.sync_copy(src_hbm.at[c, s], src_v)
    pltpu.sync_copy(dst_hbm.at[c, s], dst_v)
    if with_cnt:
        pltpu.sync_copy(z16_hbm.at[pl.ds(rbase, ROWS_PW)],
                        cntacc.at[pl.ds(rbase, ROWS_PW)])
        pltpu.sync_copy(ones_hbm, ones_v)
    plsc.subcore_barrier()

    for b in range(NBUF):
        pltpu.async_copy(p_hbm.at[src_v.at[b]], rows[b], sems[b])

    def _process(j, b, issue_next):
        pltpu.make_async_copy(p_hbm.at[src_v.at[j]], rows[b], sems[b]).wait()
        pltpu.sync_copy(rows[b], acc.at[dst_v.at[j]], add=True)
        if with_cnt:
            pltpu.sync_copy(ones_v, cntacc.at[dst_v.at[j]], add=True)
        if issue_next:
            pltpu.async_copy(p_hbm.at[src_v.at[j + NBUF]], rows[b], sems[b])

    @pl.loop(0, NCHUNK - NBUF, step=NBUF)
    def _(g):
        for b in range(NBUF):
            _process(g + b, b, True)

    for b in range(NBUF):
        _process(NCHUNK - NBUF + b, b, False)

    plsc.subcore_barrier()
    pltpu.sync_copy(acc.at[pl.ds(rbase, ROWS_PW)],
                    out_s.at[c, pl.ds(rbase, ROWS_PW)])
    if with_cnt:
        pltpu.sync_copy(cntacc.at[pl.ds(rbase, ROWS_PW)],
                        out_c.at[c, pl.ds(rbase, ROWS_PW)])


_sc_seg_sum_cnt = pl.kernel(
    functools.partial(_sc_body, True),
    out_type=[jax.ShapeDtypeStruct((NC, N_PAD, H), _f32),
              jax.ShapeDtypeStruct((NC, N_PAD, CW), _f32)],
    mesh=_sc_mesh,
    compiler_params=pltpu.CompilerParams(use_tc_tiling_on_sc=False),
    scratch_types=(
        [pltpu.VMEM_SHARED((N_PAD, H), _f32),
         pltpu.VMEM_SHARED((N_PAD, CW), _f32),
         pltpu.VMEM((NCHUNK, CHUNK), jnp.int32),
         pltpu.VMEM((NCHUNK, CHUNK), jnp.int32),
         pltpu.VMEM((CHUNK, CW), _f32)]
        + [pltpu.VMEM((CHUNK, H), _f32)] * NBUF
        + [pltpu.SemaphoreType.DMA] * NBUF
    ),
)

_sc_seg_sum = pl.kernel(
    functools.partial(_sc_body, False),
    out_type=jax.ShapeDtypeStruct((NC, N_PAD, H), _f32),
    mesh=_sc_mesh,
    compiler_params=pltpu.CompilerParams(use_tc_tiling_on_sc=False),
    scratch_types=(
        [pltpu.VMEM_SHARED((N_PAD, H), _f32),
         pltpu.VMEM((NCHUNK, CHUNK), jnp.int32),
         pltpu.VMEM((NCHUNK, CHUNK), jnp.int32)]
        + [pltpu.VMEM((CHUNK, H), _f32)] * NBUF
        + [pltpu.SemaphoreType.DMA] * NBUF
    ),
)


_BLK = 2000
_GRID = N // _BLK


def _pre_p_body(x_ref, wl_ref, p_ref):
    p_ref[...] = jnp.dot(x_ref[...], wl_ref[...], preferred_element_type=_f32)


_pre_p = pl.pallas_call(
    _pre_p_body,
    grid=(_GRID,),
    in_specs=[
        pl.BlockSpec((_BLK, D), lambda i: (i, 0)),
        pl.BlockSpec((D, H), lambda i: (0, 0)),
    ],
    out_specs=pl.BlockSpec((_BLK, H), lambda i: (i, 0)),
    out_shape=jax.ShapeDtypeStruct((N, H), _f32),
)


def _pre_rres_body(x_ref, wr_ref, wp_ref, r_ref, res_ref):
    xb = x_ref[...]
    r_ref[...] = jnp.dot(xb, wr_ref[...], preferred_element_type=_f32)
    res_ref[...] = jnp.dot(xb, wp_ref[...], preferred_element_type=_f32)


_pre_rres = pl.pallas_call(
    _pre_rres_body,
    grid=(_GRID,),
    in_specs=[
        pl.BlockSpec((_BLK, D), lambda i: (i, 0)),
        pl.BlockSpec((D, H), lambda i: (0, 0)),
        pl.BlockSpec((D, H), lambda i: (0, 0)),
    ],
    out_specs=[
        pl.BlockSpec((_BLK, H), lambda i: (i, 0)),
        pl.BlockSpec((_BLK, H), lambda i: (i, 0)),
    ],
    out_shape=[jax.ShapeDtypeStruct((N, H), _f32)] * 2,
)


def _combine(s_ref, c_ref, r_ref, bl_ref, g_ref, b_ref, res_ref):
    ssum = s_ref[0] + s_ref[1]
    cnt = c_ref[0, :, 0:1] + c_ref[1, :, 0:1]
    agg = ssum / jnp.maximum(cnt, 1.0)
    z = agg + bl_ref[...] + r_ref[...]
    mu = jnp.mean(z, axis=-1, keepdims=True)
    d = z - mu
    var = jnp.mean(d * d, axis=-1, keepdims=True)
    zn = d * lax.rsqrt(var + 1e-5) * g_ref[...] + b_ref[...]
    return jnp.maximum(zn, 0.0) + res_ref[...]


def _post_mid_body(s_ref, c_ref, r_ref, bl_ref, g_ref, b_ref, res_ref,
                   wln_ref, h_ref, pn_ref):
    h = _combine(s_ref, c_ref, r_ref, bl_ref, g_ref, b_ref, res_ref)
    h_ref[...] = h
    pn_ref[...] = jnp.dot(h, wln_ref[...], preferred_element_type=_f32)


_post_mid = pl.pallas_call(
    _post_mid_body,
    grid=(_GRID,),
    in_specs=[
        pl.BlockSpec((NC, _BLK, H), lambda i: (0, i, 0)),
        pl.BlockSpec((NC, _BLK, CW), lambda i: (0, i, 0)),
        pl.BlockSpec((_BLK, H), lambda i: (i, 0)),
        pl.BlockSpec((1, H), lambda i: (0, 0)),
        pl.BlockSpec((1, H), lambda i: (0, 0)),
        pl.BlockSpec((1, H), lambda i: (0, 0)),
        pl.BlockSpec((_BLK, H), lambda i: (i, 0)),
        pl.BlockSpec((H, H), lambda i: (0, 0)),
    ],
    out_specs=[
        pl.BlockSpec((_BLK, H), lambda i: (i, 0)),
        pl.BlockSpec((_BLK, H), lambda i: (i, 0)),
    ],
    out_shape=[jax.ShapeDtypeStruct((N, H), _f32)] * 2,
)


def _rmat_body(h_ref, w_ref, o_ref):
    o_ref[...] = jnp.dot(h_ref[...], w_ref[...], preferred_element_type=_f32)


_rmat = pl.pallas_call(
    _rmat_body,
    grid=(_GRID,),
    in_specs=[
        pl.BlockSpec((_BLK, H), lambda i: (i, 0)),
        pl.BlockSpec((H, H), lambda i: (0, 0)),
    ],
    out_specs=pl.BlockSpec((_BLK, H), lambda i: (i, 0)),
    out_shape=jax.ShapeDtypeStruct((N, H), _f32),
)


def _post_last_body(s_ref, c_ref, r_ref, bl_ref, g_ref, b_ref, res_ref,
                    wh_ref, bh_ref, h_ref, y_ref):
    h = _combine(s_ref, c_ref, r_ref, bl_ref, g_ref, b_ref, res_ref)
    h_ref[...] = h
    y_ref[...] = jnp.dot(h, wh_ref[...], preferred_element_type=_f32) + bh_ref[...]


_post_last = pl.pallas_call(
    _post_last_body,
    grid=(_GRID,),
    in_specs=[
        pl.BlockSpec((NC, _BLK, H), lambda i: (0, i, 0)),
        pl.BlockSpec((NC, _BLK, CW), lambda i: (0, i, 0)),
        pl.BlockSpec((_BLK, H), lambda i: (i, 0)),
        pl.BlockSpec((1, H), lambda i: (0, 0)),
        pl.BlockSpec((1, H), lambda i: (0, 0)),
        pl.BlockSpec((1, H), lambda i: (0, 0)),
        pl.BlockSpec((_BLK, H), lambda i: (i, 0)),
        pl.BlockSpec((H, 1), lambda i: (0, 0)),
        pl.BlockSpec((1, 1), lambda i: (0, 0)),
    ],
    out_specs=[
        pl.BlockSpec((_BLK, H), lambda i: (i, 0)),
        pl.BlockSpec((_BLK, 1), lambda i: (i, 0)),
    ],
    out_shape=[jax.ShapeDtypeStruct((N, H), _f32),
               jax.ShapeDtypeStruct((N, 1), _f32)],
)


def kernel(x, edge_index, Wl0, bl0, Wr0, ln_g0, ln_b0, Wl1, bl1, Wr1,
           ln_g1, ln_b1, Wl2, bl2, Wr2, ln_g2, ln_b2, Wproj, Whead, bhead):
    src = edge_index[0].reshape(NC, NS, NCHUNK, CHUNK)
    dst = edge_index[1].reshape(NC, NS, NCHUNK, CHUNK)
    z64 = jnp.zeros((N_PAD, H), _f32)
    z16 = jnp.zeros((N_PAD, CW), _f32)
    ones = jnp.ones((CHUNK, CW), _f32)

    bl0r, g0r, b0r = bl0.reshape(1, H), ln_g0.reshape(1, H), ln_b0.reshape(1, H)
    bl1r, g1r, b1r = bl1.reshape(1, H), ln_g1.reshape(1, H), ln_b1.reshape(1, H)
    bl2r, g2r, b2r = bl2.reshape(1, H), ln_g2.reshape(1, H), ln_b2.reshape(1, H)

    p0 = _pre_p(x, Wl0.T)
    s0, cpart = _sc_seg_sum_cnt(src, dst, p0, z64, z16, ones)
    r0, res0 = _pre_rres(x, Wr0.T, Wproj.T)
    h1, p1 = _post_mid(s0, cpart, r0, bl0r, g0r, b0r, res0, Wl1.T)
    s1 = _sc_seg_sum(src, dst, p1, z64)
    r1 = _rmat(h1, Wr1.T)
    h2, p2 = _post_mid(s1, cpart, r1, bl1r, g1r, b1r, h1, Wl2.T)
    s2 = _sc_seg_sum(src, dst, p2, z64)
    r2 = _rmat(h2, Wr2.T)
    h3, y = _post_last(s2, cpart, r2, bl2r, g2r, b2r, h2,
                       Whead.T, bhead.reshape(1, 1))
    return (y[:, 0], h3)
